# Initial kernel scaffold; baseline (speedup 1.0000x reference)
import jax
import jax.numpy as jnp
from jax.experimental import pallas as pl


def kernel(node_features, edge_index, W_in, b_in, W0, b0, g0, be0, W1, b1, g1, be1, W2, b2, g2, be2, W_out, b_out, W_op, b_op, g_op, be_op):
    raise NotImplementedError("write your pallas kernel here")


# speedup vs baseline: 2.2440x; 2.2440x over previous
import functools

import jax
import jax.numpy as jnp
from jax import lax
from jax.experimental import pallas as pl
from jax.experimental.pallas import tpu as pltpu
from jax.experimental.pallas import tpu_sc as plsc

N = 10000
E = 160000
H = 64
OUT = 64

NTILES = 32
CH = 128
NCHUNKS = E // CH
BASE_CHUNKS = NCHUNKS // NTILES
EXTRA_TILES = NCHUNKS % NTILES
DUMMY = N
NPAD = 10112
RPT = NPAD // 16
TBL = N * N

_mesh = plsc.VectorSubcoreMesh(core_axis_name="c", subcore_axis_name="s")


def _wid():
    return lax.axis_index("c") * 16 + lax.axis_index("s")


@functools.partial(
    pl.kernel,
    out_type=jax.ShapeDtypeStruct((TBL,), jnp.int32),
    mesh=_mesh,
    compiler_params=pltpu.CompilerParams(use_tc_tiling_on_sc=False),
    scratch_types=[
        pltpu.VMEM((CH,), jnp.int32),
        pltpu.VMEM((CH,), jnp.int32),
        pltpu.VMEM((CH,), jnp.int32),
        pltpu.VMEM((CH,), jnp.int32),
    ],
)
def _dedup_scatter(src_hbm, dst_hbm, table_hbm, srcb, dstb, keyb, valb):
    wid = _wid()
    lane = lax.iota(jnp.int32, 16)

    def do_chunk(k, carry):
        c = k * NTILES + wid
        off = c * CH
        pltpu.sync_copy(src_hbm.at[pl.ds(off, CH)], srcb)
        pltpu.sync_copy(dst_hbm.at[pl.ds(off, CH)], dstb)
        ebase = c * CH
        for j in range(CH // 16):
            s = srcb[pl.ds(j * 16, 16)]
            d = dstb[pl.ds(j * 16, 16)]
            keyb[pl.ds(j * 16, 16)] = s * N + d
            valb[pl.ds(j * 16, 16)] = ebase + j * 16 + lane
        pltpu.sync_copy(valb, table_hbm.at[keyb])
        return carry

    lax.fori_loop(0, BASE_CHUNKS, do_chunk, 0)

    @pl.when(wid < EXTRA_TILES)
    def _():
        do_chunk(BASE_CHUNKS, 0)


@functools.partial(
    pl.kernel,
    out_type=jax.ShapeDtypeStruct((E,), jnp.int32),
    mesh=_mesh,
    compiler_params=pltpu.CompilerParams(use_tc_tiling_on_sc=False),
    scratch_types=[
        pltpu.VMEM((CH,), jnp.int32),
        pltpu.VMEM((CH,), jnp.int32),
        pltpu.VMEM((CH,), jnp.int32),
        pltpu.VMEM((CH,), jnp.int32),
        pltpu.VMEM((CH,), jnp.int32),
        pltpu.SemaphoreType.DMA,
    ],
)
def _dedup_gather(src_hbm, dst_hbm, table_hbm, adj_hbm,
                  srcb, dstb, keyb, winb, adjb, sem):
    wid = _wid()
    lane = lax.iota(jnp.int32, 16)

    def do_chunk(k, carry):
        c = k * NTILES + wid
        off = c * CH
        pltpu.sync_copy(src_hbm.at[pl.ds(off, CH)], srcb)
        pltpu.sync_copy(dst_hbm.at[pl.ds(off, CH)], dstb)
        ebase = c * CH
        for j in range(CH // 16):
            s = srcb[pl.ds(j * 16, 16)]
            d = dstb[pl.ds(j * 16, 16)]
            keyb[pl.ds(j * 16, 16)] = s * N + d
        pltpu.async_copy(table_hbm.at[keyb], winb, sem).wait()
        for j in range(CH // 16):
            s = srcb[pl.ds(j * 16, 16)]
            w = winb[pl.ds(j * 16, 16)]
            e = ebase + j * 16 + lane
            adjb[pl.ds(j * 16, 16)] = jnp.where(w == e, s, DUMMY)
        pltpu.sync_copy(adjb, adj_hbm.at[pl.ds(off, CH)])
        return carry

    lax.fori_loop(0, BASE_CHUNKS, do_chunk, 0)

    @pl.when(wid < EXTRA_TILES)
    def _():
        do_chunk(BASE_CHUNKS, 0)


@functools.partial(
    pl.kernel,
    out_type=jax.ShapeDtypeStruct((2, NPAD, H), jnp.float32),
    mesh=_mesh,
    compiler_params=pltpu.CompilerParams(use_tc_tiling_on_sc=False),
    scratch_types=[
        pltpu.VMEM((CH,), jnp.int32),
        pltpu.VMEM((CH,), jnp.int32),
        pltpu.VMEM((CH, H), jnp.float32),
        pltpu.VMEM_SHARED((NPAD, H), jnp.float32),
        pltpu.SemaphoreType.DMA,
    ],
)
def _neighbor_sum(h_hbm, dst_hbm, adj_hbm, zrows_hbm, ns_hbm,
                  dstb, srcb, rows, acc, sem):
    core = lax.axis_index("c")
    sub = lax.axis_index("s")
    wid = core * 16 + sub

    pltpu.sync_copy(zrows_hbm, acc.at[pl.ds(sub * RPT, RPT)])
    plsc.subcore_barrier()

    def do_chunk(k, carry):
        c = k * NTILES + wid
        off = c * CH
        pltpu.sync_copy(dst_hbm.at[pl.ds(off, CH)], dstb)
        pltpu.sync_copy(adj_hbm.at[pl.ds(off, CH)], srcb)
        pltpu.async_copy(h_hbm.at[dstb], rows, sem).wait()
        pltpu.sync_copy(rows, acc.at[srcb], add=True)
        return carry

    lax.fori_loop(0, BASE_CHUNKS, do_chunk, 0)

    @pl.when(wid < EXTRA_TILES)
    def _():
        do_chunk(BASE_CHUNKS, 0)

    plsc.subcore_barrier()
    pltpu.sync_copy(acc.at[pl.ds(sub * RPT, RPT)],
                    ns_hbm.at[core, pl.ds(sub * RPT, RPT)])


def _input_proj_body(x_ref, w_ref, b_ref, o_ref):
    o_ref[...] = (jnp.dot(x_ref[...], w_ref[...],
                          preferred_element_type=jnp.float32) + b_ref[...])


def _layer_body(h_ref, ns_ref, w_ref, b_ref, g_ref, be_ref, o_ref):
    h = h_ref[...]
    ns = ns_ref[0] + ns_ref[1]
    w = w_ref[...]
    z = (jnp.dot(h, w[:H], preferred_element_type=jnp.float32)
         + jnp.dot(ns, w[H:], preferred_element_type=jnp.float32)
         + b_ref[...])
    mu = jnp.mean(z, axis=-1, keepdims=True)
    var = jnp.mean((z - mu) ** 2, axis=-1, keepdims=True)
    zn = (z - mu) * lax.rsqrt(var + 1e-5) * g_ref[...] + be_ref[...]
    o_ref[...] = jnp.maximum(zn, 0.0) + h


def _final_body(h_ref, ns_ref, w_ref, b_ref, g_ref, be_ref,
                wout_ref, bout_ref, wop_ref, bop_ref, gop_ref, beop_ref,
                o_ref):
    h = h_ref[...]
    ns = ns_ref[0] + ns_ref[1]
    w = w_ref[...]
    z = (jnp.dot(h, w[:H], preferred_element_type=jnp.float32)
         + jnp.dot(ns, w[H:], preferred_element_type=jnp.float32)
         + b_ref[...])
    mu = jnp.mean(z, axis=-1, keepdims=True)
    var = jnp.mean((z - mu) ** 2, axis=-1, keepdims=True)
    zn = (z - mu) * lax.rsqrt(var + 1e-5) * g_ref[...] + be_ref[...]
    hn = jnp.maximum(zn, 0.0) + h
    go = jnp.dot(hn, wout_ref[...], preferred_element_type=jnp.float32) + bout_ref[...]
    c = jnp.dot(go, wop_ref[...], preferred_element_type=jnp.float32) + bop_ref[...]
    mu2 = jnp.mean(c, axis=-1, keepdims=True)
    var2 = jnp.mean((c - mu2) ** 2, axis=-1, keepdims=True)
    cn = (c - mu2) * lax.rsqrt(var2 + 1e-5) * gop_ref[...] + beop_ref[...]
    o_ref[...] = jnp.maximum(cn, 0.0)


_ROWS = 1000
_GRID = N // _ROWS

_row_spec = pl.BlockSpec((_ROWS, H), lambda i: (i, 0))
_ns_spec = pl.BlockSpec((2, _ROWS, H), lambda i: (0, i, 0))
_vec_spec = pl.BlockSpec((1, H), lambda i: (0, 0))
_w_spec = pl.BlockSpec((2 * H, H), lambda i: (0, 0))
_sq_spec = pl.BlockSpec((H, H), lambda i: (0, 0))

_input_proj = pl.pallas_call(
    _input_proj_body,
    grid=(_GRID,),
    in_specs=[pl.BlockSpec((_ROWS, 8), lambda i: (i, 0)),
              pl.BlockSpec((8, H), lambda i: (0, 0)),
              _vec_spec],
    out_specs=_row_spec,
    out_shape=jax.ShapeDtypeStruct((N, H), jnp.float32),
)

_layer = pl.pallas_call(
    _layer_body,
    grid=(_GRID,),
    in_specs=[_row_spec, _ns_spec, _w_spec, _vec_spec, _vec_spec, _vec_spec],
    out_specs=_row_spec,
    out_shape=jax.ShapeDtypeStruct((N, H), jnp.float32),
)

_final = pl.pallas_call(
    _final_body,
    grid=(_GRID,),
    in_specs=[_row_spec, _ns_spec, _w_spec, _vec_spec, _vec_spec, _vec_spec,
              _sq_spec, _vec_spec, _sq_spec, _vec_spec, _vec_spec, _vec_spec],
    out_specs=_row_spec,
    out_shape=jax.ShapeDtypeStruct((N, OUT), jnp.float32),
)


def kernel(node_features, edge_index, W_in, b_in,
           W0, b0, g0, be0, W1, b1, g1, be1, W2, b2, g2, be2,
           W_out, b_out, W_op, b_op, g_op, be_op):
    src = edge_index[0]
    dst = edge_index[1]

    xp = jnp.pad(node_features, ((0, 0), (0, 2)))
    winp = jnp.pad(W_in, ((0, 2), (0, 0)))
    h = _input_proj(xp, winp, b_in.reshape(1, H))

    table = _dedup_scatter(src, dst)
    srcadj = _dedup_gather(src, dst, table)

    zrows = jnp.zeros((RPT, H), jnp.float32)
    layers = [(W0, b0, g0, be0), (W1, b1, g1, be1)]
    for (W, b, g, be) in layers:
        ns = _neighbor_sum(h, dst, srcadj, zrows)
        h = _layer(h, ns, W, b.reshape(1, H), g.reshape(1, H), be.reshape(1, H))

    ns = _neighbor_sum(h, dst, srcadj, zrows)
    out = _final(h, ns, W2, b2.reshape(1, H), g2.reshape(1, H), be2.reshape(1, H),
                 W_out, b_out.reshape(1, OUT), W_op[:OUT], b_op.reshape(1, OUT),
                 g_op.reshape(1, OUT), be_op.reshape(1, OUT))
    return out


# speedup vs baseline: 2.8969x; 1.2909x over previous
import functools

import jax
import jax.numpy as jnp
from jax import lax
from jax.experimental import pallas as pl
from jax.experimental.pallas import tpu as pltpu
from jax.experimental.pallas import tpu_sc as plsc

N = 10000
E = 160000
H = 64
OUT = 64

NTILES = 32
CH = 128
NCHUNKS = E // CH
BASE_CHUNKS = NCHUNKS // NTILES
EXTRA_TILES = NCHUNKS % NTILES
DUMMY = N
NPAD = 10112
RPT = NPAD // 16
TBL = N * N

_mesh = plsc.VectorSubcoreMesh(core_axis_name="c", subcore_axis_name="s")


def _wid():
    return lax.axis_index("c") * 16 + lax.axis_index("s")


@functools.partial(
    pl.kernel,
    out_type=jax.ShapeDtypeStruct((TBL,), jnp.int32),
    mesh=_mesh,
    compiler_params=pltpu.CompilerParams(use_tc_tiling_on_sc=False),
    scratch_types=[
        pltpu.VMEM((CH,), jnp.int32),
        pltpu.VMEM((CH,), jnp.int32),
        pltpu.VMEM((CH,), jnp.int32),
        pltpu.VMEM((CH,), jnp.int32),
    ],
)
def _dedup_scatter(src_hbm, dst_hbm, table_hbm, srcb, dstb, keyb, valb):
    wid = _wid()
    lane = lax.iota(jnp.int32, 16)

    def do_chunk(k, carry):
        c = k * NTILES + wid
        off = c * CH
        pltpu.sync_copy(src_hbm.at[pl.ds(off, CH)], srcb)
        pltpu.sync_copy(dst_hbm.at[pl.ds(off, CH)], dstb)
        ebase = c * CH
        for j in range(CH // 16):
            s = srcb[pl.ds(j * 16, 16)]
            d = dstb[pl.ds(j * 16, 16)]
            keyb[pl.ds(j * 16, 16)] = s * N + d
            valb[pl.ds(j * 16, 16)] = ebase + j * 16 + lane
        pltpu.sync_copy(valb, table_hbm.at[keyb])
        return carry

    lax.fori_loop(0, BASE_CHUNKS, do_chunk, 0)

    @pl.when(wid < EXTRA_TILES)
    def _():
        do_chunk(BASE_CHUNKS, 0)


@functools.partial(
    pl.kernel,
    out_type=jax.ShapeDtypeStruct((E,), jnp.int32),
    mesh=_mesh,
    compiler_params=pltpu.CompilerParams(use_tc_tiling_on_sc=False),
    scratch_types=[
        pltpu.VMEM((CH,), jnp.int32),
        pltpu.VMEM((CH,), jnp.int32),
        pltpu.VMEM((CH,), jnp.int32),
        pltpu.VMEM((CH,), jnp.int32),
        pltpu.VMEM((CH,), jnp.int32),
        pltpu.SemaphoreType.DMA,
    ],
)
def _dedup_gather(src_hbm, dst_hbm, table_hbm, adj_hbm,
                  srcb, dstb, keyb, winb, adjb, sem):
    wid = _wid()
    lane = lax.iota(jnp.int32, 16)

    def do_chunk(k, carry):
        c = k * NTILES + wid
        off = c * CH
        pltpu.sync_copy(src_hbm.at[pl.ds(off, CH)], srcb)
        pltpu.sync_copy(dst_hbm.at[pl.ds(off, CH)], dstb)
        ebase = c * CH
        for j in range(CH // 16):
            s = srcb[pl.ds(j * 16, 16)]
            d = dstb[pl.ds(j * 16, 16)]
            keyb[pl.ds(j * 16, 16)] = s * N + d
        pltpu.async_copy(table_hbm.at[keyb], winb, sem).wait()
        for j in range(CH // 16):
            s = srcb[pl.ds(j * 16, 16)]
            w = winb[pl.ds(j * 16, 16)]
            e = ebase + j * 16 + lane
            adjb[pl.ds(j * 16, 16)] = jnp.where(w == e, s, DUMMY)
        pltpu.sync_copy(adjb, adj_hbm.at[pl.ds(off, CH)])
        return carry

    lax.fori_loop(0, BASE_CHUNKS, do_chunk, 0)

    @pl.when(wid < EXTRA_TILES)
    def _():
        do_chunk(BASE_CHUNKS, 0)


@functools.partial(
    pl.kernel,
    out_type=jax.ShapeDtypeStruct((2, NPAD, H), jnp.float32),
    mesh=_mesh,
    compiler_params=pltpu.CompilerParams(use_tc_tiling_on_sc=False),
    scratch_types=[
        pltpu.VMEM((3, CH), jnp.int32),
        pltpu.VMEM((3, CH), jnp.int32),
        pltpu.VMEM((3, CH, H), jnp.float32),
        pltpu.VMEM_SHARED((NPAD, H), jnp.float32),
        pltpu.SemaphoreType.DMA((3,)),
        pltpu.SemaphoreType.DMA((3,)),
        pltpu.SemaphoreType.DMA((3,)),
        pltpu.SemaphoreType.DMA((3,)),
    ],
)
def _neighbor_sum(h_hbm, dst_hbm, adj_hbm, zrows_hbm, ns_hbm,
                  dstb, srcb, rows, acc, semd, sema, semg, semsc):
    core = lax.axis_index("c")
    sub = lax.axis_index("s")
    wid = core * 16 + sub

    pltpu.sync_copy(zrows_hbm, acc.at[pl.ds(sub * RPT, RPT)])
    plsc.subcore_barrier()

    def off_of(k):
        return (k * NTILES + wid) * CH

    def idx_start(k, sl):
        off = off_of(k)
        pltpu.async_copy(dst_hbm.at[pl.ds(off, CH)], dstb.at[sl], semd.at[sl])
        pltpu.async_copy(adj_hbm.at[pl.ds(off, CH)], srcb.at[sl], sema.at[sl])

    def idx_wait(k, sl):
        off = off_of(k)
        pltpu.make_async_copy(dst_hbm.at[pl.ds(off, CH)], dstb.at[sl],
                              semd.at[sl]).wait()
        pltpu.make_async_copy(adj_hbm.at[pl.ds(off, CH)], srcb.at[sl],
                              sema.at[sl]).wait()

    def scat_wait(sl):
        pltpu.make_async_copy(rows.at[sl], acc.at[srcb.at[sl]],
                              semsc.at[sl]).wait()

    def step(k, sl, waitprev, knext):
        slnext = (sl + 1) % 3
        if waitprev:
            scat_wait(slnext)
        if knext is not None:
            idx_start(knext, slnext)
        idx_wait(k, sl)
        pltpu.async_copy(h_hbm.at[dstb.at[sl]], rows.at[sl],
                         semg.at[sl]).wait()
        pltpu.async_copy(rows.at[sl], acc.at[srcb.at[sl]], semsc.at[sl],
                         add=True)

    assert BASE_CHUNKS == 39
    idx_start(0, 0)
    step(0, 0, False, 1)
    step(1, 1, False, 2)

    def grp(g, carry):
        k0 = 2 + 3 * g
        step(k0, 2, True, k0 + 1)
        step(k0 + 1, 0, True, k0 + 2)
        step(k0 + 2, 1, True, k0 + 3)
        return carry

    lax.fori_loop(0, 12, grp, 0)
    step(38, 2, True, None)
    scat_wait(1)
    scat_wait(2)

    @pl.when(wid < EXTRA_TILES)
    def _():
        idx_start(BASE_CHUNKS, 0)
        idx_wait(BASE_CHUNKS, 0)
        pltpu.async_copy(h_hbm.at[dstb.at[0]], rows.at[0], semg.at[0]).wait()
        pltpu.async_copy(rows.at[0], acc.at[srcb.at[0]], semsc.at[0],
                         add=True)
        scat_wait(0)

    plsc.subcore_barrier()
    pltpu.sync_copy(acc.at[pl.ds(sub * RPT, RPT)],
                    ns_hbm.at[core, pl.ds(sub * RPT, RPT)])


def _input_proj_body(x_ref, w_ref, b_ref, o_ref):
    o_ref[...] = (jnp.dot(x_ref[...], w_ref[...],
                          preferred_element_type=jnp.float32) + b_ref[...])


def _layer_body(h_ref, ns_ref, w_ref, b_ref, g_ref, be_ref, o_ref):
    h = h_ref[...]
    ns = ns_ref[0] + ns_ref[1]
    w = w_ref[...]
    z = (jnp.dot(h, w[:H], preferred_element_type=jnp.float32)
         + jnp.dot(ns, w[H:], preferred_element_type=jnp.float32)
         + b_ref[...])
    mu = jnp.mean(z, axis=-1, keepdims=True)
    var = jnp.mean((z - mu) ** 2, axis=-1, keepdims=True)
    zn = (z - mu) * lax.rsqrt(var + 1e-5) * g_ref[...] + be_ref[...]
    o_ref[...] = jnp.maximum(zn, 0.0) + h


def _final_body(h_ref, ns_ref, w_ref, b_ref, g_ref, be_ref,
                wout_ref, bout_ref, wop_ref, bop_ref, gop_ref, beop_ref,
                o_ref):
    h = h_ref[...]
    ns = ns_ref[0] + ns_ref[1]
    w = w_ref[...]
    z = (jnp.dot(h, w[:H], preferred_element_type=jnp.float32)
         + jnp.dot(ns, w[H:], preferred_element_type=jnp.float32)
         + b_ref[...])
    mu = jnp.mean(z, axis=-1, keepdims=True)
    var = jnp.mean((z - mu) ** 2, axis=-1, keepdims=True)
    zn = (z - mu) * lax.rsqrt(var + 1e-5) * g_ref[...] + be_ref[...]
    hn = jnp.maximum(zn, 0.0) + h
    go = jnp.dot(hn, wout_ref[...], preferred_element_type=jnp.float32) + bout_ref[...]
    c = jnp.dot(go, wop_ref[...], preferred_element_type=jnp.float32) + bop_ref[...]
    mu2 = jnp.mean(c, axis=-1, keepdims=True)
    var2 = jnp.mean((c - mu2) ** 2, axis=-1, keepdims=True)
    cn = (c - mu2) * lax.rsqrt(var2 + 1e-5) * gop_ref[...] + beop_ref[...]
    o_ref[...] = jnp.maximum(cn, 0.0)


_ROWS = 1000
_GRID = N // _ROWS

_row_spec = pl.BlockSpec((_ROWS, H), lambda i: (i, 0))
_ns_spec = pl.BlockSpec((2, _ROWS, H), lambda i: (0, i, 0))
_vec_spec = pl.BlockSpec((1, H), lambda i: (0, 0))
_w_spec = pl.BlockSpec((2 * H, H), lambda i: (0, 0))
_sq_spec = pl.BlockSpec((H, H), lambda i: (0, 0))

_input_proj = pl.pallas_call(
    _input_proj_body,
    grid=(_GRID,),
    in_specs=[pl.BlockSpec((_ROWS, 8), lambda i: (i, 0)),
              pl.BlockSpec((8, H), lambda i: (0, 0)),
              _vec_spec],
    out_specs=_row_spec,
    out_shape=jax.ShapeDtypeStruct((N, H), jnp.float32),
)

_layer = pl.pallas_call(
    _layer_body,
    grid=(_GRID,),
    in_specs=[_row_spec, _ns_spec, _w_spec, _vec_spec, _vec_spec, _vec_spec],
    out_specs=_row_spec,
    out_shape=jax.ShapeDtypeStruct((N, H), jnp.float32),
)

_final = pl.pallas_call(
    _final_body,
    grid=(_GRID,),
    in_specs=[_row_spec, _ns_spec, _w_spec, _vec_spec, _vec_spec, _vec_spec,
              _sq_spec, _vec_spec, _sq_spec, _vec_spec, _vec_spec, _vec_spec],
    out_specs=_row_spec,
    out_shape=jax.ShapeDtypeStruct((N, OUT), jnp.float32),
)


def kernel(node_features, edge_index, W_in, b_in,
           W0, b0, g0, be0, W1, b1, g1, be1, W2, b2, g2, be2,
           W_out, b_out, W_op, b_op, g_op, be_op):
    src = edge_index[0]
    dst = edge_index[1]

    xp = jnp.pad(node_features, ((0, 0), (0, 2)))
    winp = jnp.pad(W_in, ((0, 2), (0, 0)))
    h = _input_proj(xp, winp, b_in.reshape(1, H))

    table = _dedup_scatter(src, dst)
    srcadj = _dedup_gather(src, dst, table)

    zrows = jnp.zeros((RPT, H), jnp.float32)
    layers = [(W0, b0, g0, be0), (W1, b1, g1, be1)]
    for (W, b, g, be) in layers:
        ns = _neighbor_sum(h, dst, srcadj, zrows)
        h = _layer(h, ns, W, b.reshape(1, H), g.reshape(1, H), be.reshape(1, H))

    ns = _neighbor_sum(h, dst, srcadj, zrows)
    out = _final(h, ns, W2, b2.reshape(1, H), g2.reshape(1, H), be2.reshape(1, H),
                 W_out, b_out.reshape(1, OUT), W_op[:OUT], b_op.reshape(1, OUT),
                 g_op.reshape(1, OUT), be_op.reshape(1, OUT))
    return out


# speedup vs baseline: 3.2122x; 1.1088x over previous
import functools

import jax
import jax.numpy as jnp
from jax import lax
from jax.experimental import pallas as pl
from jax.experimental.pallas import tpu as pltpu
from jax.experimental.pallas import tpu_sc as plsc

N = 10000
E = 160000
H = 64
OUT = 64

NTILES = 32
CH = 128
NCHUNKS = E // CH
BASE_CHUNKS = NCHUNKS // NTILES
EXTRA_TILES = NCHUNKS % NTILES
DUMMY = N
NPAD = 10112
RPT = NPAD // 16
TBL = N * N

_mesh = plsc.VectorSubcoreMesh(core_axis_name="c", subcore_axis_name="s")


def _wid():
    return lax.axis_index("c") * 16 + lax.axis_index("s")


@functools.partial(
    pl.kernel,
    out_type=jax.ShapeDtypeStruct((TBL,), jnp.int32),
    mesh=_mesh,
    compiler_params=pltpu.CompilerParams(use_tc_tiling_on_sc=False),
    scratch_types=[
        pltpu.VMEM((3, CH), jnp.int32),
        pltpu.VMEM((3, CH), jnp.int32),
        pltpu.VMEM((3, CH), jnp.int32),
        pltpu.VMEM((3, CH), jnp.int32),
        pltpu.SemaphoreType.DMA((3,)),
        pltpu.SemaphoreType.DMA((3,)),
        pltpu.SemaphoreType.DMA((3,)),
    ],
)
def _dedup_scatter(src_hbm, dst_hbm, table_hbm,
                   srcb, dstb, keyb, valb, semd, sema, semsc):
    wid = _wid()
    lane = lax.iota(jnp.int32, 16)

    def off_of(k):
        return (k * NTILES + wid) * CH

    def idx_start(k, sl):
        off = off_of(k)
        pltpu.async_copy(src_hbm.at[pl.ds(off, CH)], srcb.at[sl], semd.at[sl])
        pltpu.async_copy(dst_hbm.at[pl.ds(off, CH)], dstb.at[sl], sema.at[sl])

    def idx_wait(k, sl):
        off = off_of(k)
        pltpu.make_async_copy(src_hbm.at[pl.ds(off, CH)], srcb.at[sl],
                              semd.at[sl]).wait()
        pltpu.make_async_copy(dst_hbm.at[pl.ds(off, CH)], dstb.at[sl],
                              sema.at[sl]).wait()

    def scat_wait(sl):
        pltpu.make_async_copy(valb.at[sl], table_hbm.at[keyb.at[sl]],
                              semsc.at[sl]).wait()

    def step(k, sl, waitprev, knext):
        slnext = (sl + 1) % 3
        if waitprev:
            scat_wait(slnext)
        if knext is not None:
            idx_start(knext, slnext)
        idx_wait(k, sl)
        ebase = (k * NTILES + wid) * CH
        for j in range(CH // 16):
            s = srcb[sl, pl.ds(j * 16, 16)]
            d = dstb[sl, pl.ds(j * 16, 16)]
            keyb[sl, pl.ds(j * 16, 16)] = s * N + d
            valb[sl, pl.ds(j * 16, 16)] = ebase + j * 16 + lane
        pltpu.async_copy(valb.at[sl], table_hbm.at[keyb.at[sl]], semsc.at[sl])

    assert BASE_CHUNKS == 39
    idx_start(0, 0)
    step(0, 0, False, 1)
    step(1, 1, False, 2)

    def grp(g, carry):
        k0 = 2 + 3 * g
        step(k0, 2, True, k0 + 1)
        step(k0 + 1, 0, True, k0 + 2)
        step(k0 + 2, 1, True, k0 + 3)
        return carry

    lax.fori_loop(0, 12, grp, 0)
    step(38, 2, True, None)
    scat_wait(1)
    scat_wait(2)

    @pl.when(wid < EXTRA_TILES)
    def _():
        idx_start(BASE_CHUNKS, 0)
        step(BASE_CHUNKS, 0, False, None)
        scat_wait(0)


@functools.partial(
    pl.kernel,
    out_type=jax.ShapeDtypeStruct((E,), jnp.int32),
    mesh=_mesh,
    compiler_params=pltpu.CompilerParams(use_tc_tiling_on_sc=False),
    scratch_types=[
        pltpu.VMEM((3, CH), jnp.int32),
        pltpu.VMEM((3, CH), jnp.int32),
        pltpu.VMEM((3, CH), jnp.int32),
        pltpu.VMEM((3, CH), jnp.int32),
        pltpu.VMEM((3, CH), jnp.int32),
        pltpu.SemaphoreType.DMA((3,)),
        pltpu.SemaphoreType.DMA((3,)),
        pltpu.SemaphoreType.DMA((3,)),
        pltpu.SemaphoreType.DMA((3,)),
    ],
)
def _dedup_gather(src_hbm, dst_hbm, table_hbm, adj_hbm,
                  srcb, dstb, keyb, winb, adjb, semd, sema, semg, semst):
    wid = _wid()
    lane = lax.iota(jnp.int32, 16)

    def off_of(k):
        return (k * NTILES + wid) * CH

    def idx_start(k, sl):
        off = off_of(k)
        pltpu.async_copy(src_hbm.at[pl.ds(off, CH)], srcb.at[sl], semd.at[sl])
        pltpu.async_copy(dst_hbm.at[pl.ds(off, CH)], dstb.at[sl], sema.at[sl])

    def idx_wait(k, sl):
        off = off_of(k)
        pltpu.make_async_copy(src_hbm.at[pl.ds(off, CH)], srcb.at[sl],
                              semd.at[sl]).wait()
        pltpu.make_async_copy(dst_hbm.at[pl.ds(off, CH)], dstb.at[sl],
                              sema.at[sl]).wait()

    def store_wait(k, sl):
        pltpu.make_async_copy(adjb.at[sl], adj_hbm.at[pl.ds(off_of(k), CH)],
                              semst.at[sl]).wait()

    def front(k, sl):
        idx_wait(k, sl)
        for j in range(CH // 16):
            s = srcb[sl, pl.ds(j * 16, 16)]
            d = dstb[sl, pl.ds(j * 16, 16)]
            keyb[sl, pl.ds(j * 16, 16)] = s * N + d
        pltpu.async_copy(table_hbm.at[keyb.at[sl]], winb.at[sl], semg.at[sl])

    def back(k, sl):
        pltpu.make_async_copy(table_hbm.at[keyb.at[sl]], winb.at[sl],
                              semg.at[sl]).wait()
        ebase = (k * NTILES + wid) * CH
        for j in range(CH // 16):
            s = srcb[sl, pl.ds(j * 16, 16)]
            w = winb[sl, pl.ds(j * 16, 16)]
            e = ebase + j * 16 + lane
            adjb[sl, pl.ds(j * 16, 16)] = jnp.where(w == e, s, DUMMY)
        pltpu.async_copy(adjb.at[sl], adj_hbm.at[pl.ds(ebase, CH)],
                         semst.at[sl])

    assert BASE_CHUNKS == 39
    idx_start(0, 0)
    idx_start(1, 1)
    idx_start(2, 2)
    front(0, 0)
    front(1, 1)
    back(0, 0)

    def grp(g, carry):
        k0 = 2 + 3 * g
        for u, sl in ((0, 2), (1, 0), (2, 1)):
            k = k0 + u
            slnext = (sl + 1) % 3
            store_wait(k - 2, slnext)
            idx_start(k + 1, slnext)
            front(k, sl)
            back(k - 1, (sl + 2) % 3)
        return carry

    lax.fori_loop(0, 12, grp, 0)
    store_wait(36, 0)
    front(38, 2)
    back(37, 1)
    back(38, 2)
    store_wait(37, 1)
    store_wait(38, 2)

    @pl.when(wid < EXTRA_TILES)
    def _():
        idx_start(BASE_CHUNKS, 0)
        front(BASE_CHUNKS, 0)
        back(BASE_CHUNKS, 0)
        store_wait(BASE_CHUNKS, 0)


@functools.partial(
    pl.kernel,
    out_type=jax.ShapeDtypeStruct((2, NPAD, H), jnp.float32),
    mesh=_mesh,
    compiler_params=pltpu.CompilerParams(use_tc_tiling_on_sc=False),
    scratch_types=[
        pltpu.VMEM((3, CH), jnp.int32),
        pltpu.VMEM((3, CH), jnp.int32),
        pltpu.VMEM((3, CH, H), jnp.float32),
        pltpu.VMEM_SHARED((NPAD, H), jnp.float32),
        pltpu.SemaphoreType.DMA((3,)),
        pltpu.SemaphoreType.DMA((3,)),
        pltpu.SemaphoreType.DMA((3,)),
        pltpu.SemaphoreType.DMA((3,)),
    ],
)
def _neighbor_sum(h_hbm, dst_hbm, adj_hbm, zrows_hbm, ns_hbm,
                  dstb, srcb, rows, acc, semd, sema, semg, semsc):
    core = lax.axis_index("c")
    sub = lax.axis_index("s")
    wid = core * 16 + sub

    pltpu.sync_copy(zrows_hbm, acc.at[pl.ds(sub * RPT, RPT)])
    plsc.subcore_barrier()

    def off_of(k):
        return (k * NTILES + wid) * CH

    def idx_start(k, sl):
        off = off_of(k)
        pltpu.async_copy(dst_hbm.at[pl.ds(off, CH)], dstb.at[sl], semd.at[sl])
        pltpu.async_copy(adj_hbm.at[pl.ds(off, CH)], srcb.at[sl], sema.at[sl])

    def idx_wait(k, sl):
        off = off_of(k)
        pltpu.make_async_copy(dst_hbm.at[pl.ds(off, CH)], dstb.at[sl],
                              semd.at[sl]).wait()
        pltpu.make_async_copy(adj_hbm.at[pl.ds(off, CH)], srcb.at[sl],
                              sema.at[sl]).wait()

    def scat_wait(sl):
        pltpu.make_async_copy(rows.at[sl], acc.at[srcb.at[sl]],
                              semsc.at[sl]).wait()

    def step(k, sl, waitprev, knext):
        slnext = (sl + 1) % 3
        if waitprev:
            scat_wait(slnext)
        if knext is not None:
            idx_start(knext, slnext)
        idx_wait(k, sl)
        pltpu.async_copy(h_hbm.at[dstb.at[sl]], rows.at[sl],
                         semg.at[sl]).wait()
        pltpu.async_copy(rows.at[sl], acc.at[srcb.at[sl]], semsc.at[sl],
                         add=True)

    assert BASE_CHUNKS == 39
    idx_start(0, 0)
    step(0, 0, False, 1)
    step(1, 1, False, 2)

    def grp(g, carry):
        k0 = 2 + 3 * g
        step(k0, 2, True, k0 + 1)
        step(k0 + 1, 0, True, k0 + 2)
        step(k0 + 2, 1, True, k0 + 3)
        return carry

    lax.fori_loop(0, 12, grp, 0)
    step(38, 2, True, None)
    scat_wait(1)
    scat_wait(2)

    @pl.when(wid < EXTRA_TILES)
    def _():
        idx_start(BASE_CHUNKS, 0)
        idx_wait(BASE_CHUNKS, 0)
        pltpu.async_copy(h_hbm.at[dstb.at[0]], rows.at[0], semg.at[0]).wait()
        pltpu.async_copy(rows.at[0], acc.at[srcb.at[0]], semsc.at[0],
                         add=True)
        scat_wait(0)

    plsc.subcore_barrier()
    pltpu.sync_copy(acc.at[pl.ds(sub * RPT, RPT)],
                    ns_hbm.at[core, pl.ds(sub * RPT, RPT)])


def _input_proj_body(x_ref, w_ref, b_ref, o_ref):
    o_ref[...] = (jnp.dot(x_ref[...], w_ref[...],
                          preferred_element_type=jnp.float32) + b_ref[...])


def _layer_body(h_ref, ns_ref, w_ref, b_ref, g_ref, be_ref, o_ref):
    h = h_ref[...]
    ns = ns_ref[0] + ns_ref[1]
    w = w_ref[...]
    z = (jnp.dot(h, w[:H], preferred_element_type=jnp.float32)
         + jnp.dot(ns, w[H:], preferred_element_type=jnp.float32)
         + b_ref[...])
    mu = jnp.mean(z, axis=-1, keepdims=True)
    var = jnp.mean((z - mu) ** 2, axis=-1, keepdims=True)
    zn = (z - mu) * lax.rsqrt(var + 1e-5) * g_ref[...] + be_ref[...]
    o_ref[...] = jnp.maximum(zn, 0.0) + h


def _final_body(h_ref, ns_ref, w_ref, b_ref, g_ref, be_ref,
                wout_ref, bout_ref, wop_ref, bop_ref, gop_ref, beop_ref,
                o_ref):
    h = h_ref[...]
    ns = ns_ref[0] + ns_ref[1]
    w = w_ref[...]
    z = (jnp.dot(h, w[:H], preferred_element_type=jnp.float32)
         + jnp.dot(ns, w[H:], preferred_element_type=jnp.float32)
         + b_ref[...])
    mu = jnp.mean(z, axis=-1, keepdims=True)
    var = jnp.mean((z - mu) ** 2, axis=-1, keepdims=True)
    zn = (z - mu) * lax.rsqrt(var + 1e-5) * g_ref[...] + be_ref[...]
    hn = jnp.maximum(zn, 0.0) + h
    go = jnp.dot(hn, wout_ref[...], preferred_element_type=jnp.float32) + bout_ref[...]
    c = jnp.dot(go, wop_ref[...], preferred_element_type=jnp.float32) + bop_ref[...]
    mu2 = jnp.mean(c, axis=-1, keepdims=True)
    var2 = jnp.mean((c - mu2) ** 2, axis=-1, keepdims=True)
    cn = (c - mu2) * lax.rsqrt(var2 + 1e-5) * gop_ref[...] + beop_ref[...]
    o_ref[...] = jnp.maximum(cn, 0.0)


_ROWS = 1000
_GRID = N // _ROWS

_row_spec = pl.BlockSpec((_ROWS, H), lambda i: (i, 0))
_ns_spec = pl.BlockSpec((2, _ROWS, H), lambda i: (0, i, 0))
_vec_spec = pl.BlockSpec((1, H), lambda i: (0, 0))
_w_spec = pl.BlockSpec((2 * H, H), lambda i: (0, 0))
_sq_spec = pl.BlockSpec((H, H), lambda i: (0, 0))

_input_proj = pl.pallas_call(
    _input_proj_body,
    grid=(_GRID,),
    in_specs=[pl.BlockSpec((_ROWS, 8), lambda i: (i, 0)),
              pl.BlockSpec((8, H), lambda i: (0, 0)),
              _vec_spec],
    out_specs=_row_spec,
    out_shape=jax.ShapeDtypeStruct((N, H), jnp.float32),
)

_layer = pl.pallas_call(
    _layer_body,
    grid=(_GRID,),
    in_specs=[_row_spec, _ns_spec, _w_spec, _vec_spec, _vec_spec, _vec_spec],
    out_specs=_row_spec,
    out_shape=jax.ShapeDtypeStruct((N, H), jnp.float32),
)

_final = pl.pallas_call(
    _final_body,
    grid=(_GRID,),
    in_specs=[_row_spec, _ns_spec, _w_spec, _vec_spec, _vec_spec, _vec_spec,
              _sq_spec, _vec_spec, _sq_spec, _vec_spec, _vec_spec, _vec_spec],
    out_specs=_row_spec,
    out_shape=jax.ShapeDtypeStruct((N, OUT), jnp.float32),
)


def kernel(node_features, edge_index, W_in, b_in,
           W0, b0, g0, be0, W1, b1, g1, be1, W2, b2, g2, be2,
           W_out, b_out, W_op, b_op, g_op, be_op):
    src = edge_index[0]
    dst = edge_index[1]

    xp = jnp.pad(node_features, ((0, 0), (0, 2)))
    winp = jnp.pad(W_in, ((0, 2), (0, 0)))
    h = _input_proj(xp, winp, b_in.reshape(1, H))

    table = _dedup_scatter(src, dst)
    srcadj = _dedup_gather(src, dst, table)

    zrows = jnp.zeros((RPT, H), jnp.float32)
    layers = [(W0, b0, g0, be0), (W1, b1, g1, be1)]
    for (W, b, g, be) in layers:
        ns = _neighbor_sum(h, dst, srcadj, zrows)
        h = _layer(h, ns, W, b.reshape(1, H), g.reshape(1, H), be.reshape(1, H))

    ns = _neighbor_sum(h, dst, srcadj, zrows)
    out = _final(h, ns, W2, b2.reshape(1, H), g2.reshape(1, H), be2.reshape(1, H),
                 W_out, b_out.reshape(1, OUT), W_op[:OUT], b_op.reshape(1, OUT),
                 g_op.reshape(1, OUT), be_op.reshape(1, OUT))
    return out


# speedup vs baseline: 4.6744x; 1.4552x over previous
import functools

import jax
import jax.numpy as jnp
from jax import lax
from jax.experimental import pallas as pl
from jax.experimental.pallas import tpu as pltpu
from jax.experimental.pallas import tpu_sc as plsc

N = 10000
E = 160000
H = 64
OUT = 64

NTILES = 32
CH = 128
NCHUNKS = E // CH
BASE_CHUNKS = NCHUNKS // NTILES
EXTRA_TILES = NCHUNKS % NTILES
DUMMY = N
NPAD = 10112
RPT = NPAD // 16
TBL = N * N

_mesh = plsc.VectorSubcoreMesh(core_axis_name="c", subcore_axis_name="s")


def _wid():
    return lax.axis_index("c") * 16 + lax.axis_index("s")


@functools.partial(
    pl.kernel,
    out_type=jax.ShapeDtypeStruct((TBL, 16), jnp.int32),
    mesh=_mesh,
    compiler_params=pltpu.CompilerParams(use_tc_tiling_on_sc=False, needs_layout_passes=False),
    scratch_types=[
        pltpu.VMEM((3, CH), jnp.int32),
        pltpu.VMEM((3, CH), jnp.int32),
        pltpu.VMEM((3, CH), jnp.int32),
        pltpu.VMEM((3, CH, 16), jnp.int32),
        pltpu.SemaphoreType.DMA((3,)),
        pltpu.SemaphoreType.DMA((3,)),
        pltpu.SemaphoreType.DMA((3,)),
    ],
)
def _dedup_scatter(src_hbm, dst_hbm, table_hbm,
                   srcb, dstb, keyb, valb, semd, sema, semsc):
    wid = _wid()
    lane = lax.iota(jnp.int32, 16)

    def off_of(k):
        return (k * NTILES + wid) * CH

    def idx_start(k, sl):
        off = off_of(k)
        pltpu.async_copy(src_hbm.at[pl.ds(off, CH)], srcb.at[sl], semd.at[sl])
        pltpu.async_copy(dst_hbm.at[pl.ds(off, CH)], dstb.at[sl], sema.at[sl])

    def idx_wait(k, sl):
        off = off_of(k)
        pltpu.make_async_copy(src_hbm.at[pl.ds(off, CH)], srcb.at[sl],
                              semd.at[sl]).wait()
        pltpu.make_async_copy(dst_hbm.at[pl.ds(off, CH)], dstb.at[sl],
                              sema.at[sl]).wait()

    def scat_wait(sl):
        pltpu.make_async_copy(valb.at[sl], table_hbm.at[keyb.at[sl]],
                              semsc.at[sl]).wait()

    zero16 = jnp.zeros((16,), jnp.int32)

    def step(k, sl, waitprev, knext):
        slnext = (sl + 1) % 3
        if waitprev:
            scat_wait(slnext)
        if knext is not None:
            idx_start(knext, slnext)
        idx_wait(k, sl)
        ebase = (k * NTILES + wid) * CH
        for j in range(CH // 16):
            s = srcb[sl, pl.ds(j * 16, 16)]
            d = dstb[sl, pl.ds(j * 16, 16)]
            keyb[sl, pl.ds(j * 16, 16)] = s * N + d
            plsc.store_scatter(valb.at[sl], [j * 16 + lane, zero16],
                               ebase + j * 16 + lane)
        pltpu.async_copy(valb.at[sl], table_hbm.at[keyb.at[sl]], semsc.at[sl])

    assert BASE_CHUNKS == 39
    idx_start(0, 0)
    step(0, 0, False, 1)
    step(1, 1, False, 2)

    def grp(g, carry):
        k0 = 2 + 3 * g
        step(k0, 2, True, k0 + 1)
        step(k0 + 1, 0, True, k0 + 2)
        step(k0 + 2, 1, True, k0 + 3)
        return carry

    lax.fori_loop(0, 12, grp, 0)
    step(38, 2, True, None)
    scat_wait(1)
    scat_wait(2)

    @pl.when(wid < EXTRA_TILES)
    def _():
        idx_start(BASE_CHUNKS, 0)
        step(BASE_CHUNKS, 0, False, None)
        scat_wait(0)


@functools.partial(
    pl.kernel,
    out_type=jax.ShapeDtypeStruct((E,), jnp.int32),
    mesh=_mesh,
    compiler_params=pltpu.CompilerParams(use_tc_tiling_on_sc=False, needs_layout_passes=False),
    scratch_types=[
        pltpu.VMEM((3, CH), jnp.int32),
        pltpu.VMEM((3, CH), jnp.int32),
        pltpu.VMEM((3, CH), jnp.int32),
        pltpu.VMEM((3, CH, 16), jnp.int32),
        pltpu.VMEM((3, CH), jnp.int32),
        pltpu.SemaphoreType.DMA((3,)),
        pltpu.SemaphoreType.DMA((3,)),
        pltpu.SemaphoreType.DMA((3,)),
        pltpu.SemaphoreType.DMA((3,)),
    ],
)
def _dedup_gather(src_hbm, dst_hbm, table_hbm, adj_hbm,
                  srcb, dstb, keyb, winb, adjb, semd, sema, semg, semst):
    wid = _wid()
    lane = lax.iota(jnp.int32, 16)

    def off_of(k):
        return (k * NTILES + wid) * CH

    def idx_start(k, sl):
        off = off_of(k)
        pltpu.async_copy(src_hbm.at[pl.ds(off, CH)], srcb.at[sl], semd.at[sl])
        pltpu.async_copy(dst_hbm.at[pl.ds(off, CH)], dstb.at[sl], sema.at[sl])

    def idx_wait(k, sl):
        off = off_of(k)
        pltpu.make_async_copy(src_hbm.at[pl.ds(off, CH)], srcb.at[sl],
                              semd.at[sl]).wait()
        pltpu.make_async_copy(dst_hbm.at[pl.ds(off, CH)], dstb.at[sl],
                              sema.at[sl]).wait()

    def store_wait(k, sl):
        pltpu.make_async_copy(adjb.at[sl], adj_hbm.at[pl.ds(off_of(k), CH)],
                              semst.at[sl]).wait()

    def front(k, sl):
        idx_wait(k, sl)
        for j in range(CH // 16):
            s = srcb[sl, pl.ds(j * 16, 16)]
            d = dstb[sl, pl.ds(j * 16, 16)]
            keyb[sl, pl.ds(j * 16, 16)] = s * N + d
        pltpu.async_copy(table_hbm.at[keyb.at[sl]], winb.at[sl], semg.at[sl])

    zero16 = jnp.zeros((16,), jnp.int32)

    def back(k, sl):
        pltpu.make_async_copy(table_hbm.at[keyb.at[sl]], winb.at[sl],
                              semg.at[sl]).wait()
        ebase = (k * NTILES + wid) * CH
        for j in range(CH // 16):
            s = srcb[sl, pl.ds(j * 16, 16)]
            w = plsc.load_gather(winb.at[sl], [j * 16 + lane, zero16])
            e = ebase + j * 16 + lane
            adjb[sl, pl.ds(j * 16, 16)] = jnp.where(w == e, s, DUMMY)
        pltpu.async_copy(adjb.at[sl], adj_hbm.at[pl.ds(ebase, CH)],
                         semst.at[sl])

    assert BASE_CHUNKS == 39
    idx_start(0, 0)
    idx_start(1, 1)
    idx_start(2, 2)
    front(0, 0)
    front(1, 1)
    back(0, 0)

    def grp(g, carry):
        k0 = 2 + 3 * g
        for u, sl in ((0, 2), (1, 0), (2, 1)):
            k = k0 + u
            slnext = (sl + 1) % 3
            store_wait(k - 2, slnext)
            idx_start(k + 1, slnext)
            front(k, sl)
            back(k - 1, (sl + 2) % 3)
        return carry

    lax.fori_loop(0, 12, grp, 0)
    store_wait(36, 0)
    front(38, 2)
    back(37, 1)
    back(38, 2)
    store_wait(37, 1)
    store_wait(38, 2)

    @pl.when(wid < EXTRA_TILES)
    def _():
        idx_start(BASE_CHUNKS, 0)
        front(BASE_CHUNKS, 0)
        back(BASE_CHUNKS, 0)
        store_wait(BASE_CHUNKS, 0)


@functools.partial(
    pl.kernel,
    out_type=jax.ShapeDtypeStruct((2, NPAD, H), jnp.float32),
    mesh=_mesh,
    compiler_params=pltpu.CompilerParams(use_tc_tiling_on_sc=False, needs_layout_passes=False),
    scratch_types=[
        pltpu.VMEM((3, CH), jnp.int32),
        pltpu.VMEM((3, CH), jnp.int32),
        pltpu.VMEM((3, CH, H), jnp.float32),
        pltpu.VMEM_SHARED((NPAD, H), jnp.float32),
        pltpu.SemaphoreType.DMA((3,)),
        pltpu.SemaphoreType.DMA((3,)),
        pltpu.SemaphoreType.DMA((3,)),
        pltpu.SemaphoreType.DMA((3,)),
    ],
)
def _neighbor_sum(h_hbm, dst_hbm, adj_hbm, zrows_hbm, ns_hbm,
                  dstb, srcb, rows, acc, semd, sema, semg, semsc):
    core = lax.axis_index("c")
    sub = lax.axis_index("s")
    wid = core * 16 + sub

    pltpu.sync_copy(zrows_hbm, acc.at[pl.ds(sub * RPT, RPT)])
    plsc.subcore_barrier()

    def off_of(k):
        return (k * NTILES + wid) * CH

    def idx_start(k, sl):
        off = off_of(k)
        pltpu.async_copy(dst_hbm.at[pl.ds(off, CH)], dstb.at[sl], semd.at[sl])
        pltpu.async_copy(adj_hbm.at[pl.ds(off, CH)], srcb.at[sl], sema.at[sl])

    def idx_wait(k, sl):
        off = off_of(k)
        pltpu.make_async_copy(dst_hbm.at[pl.ds(off, CH)], dstb.at[sl],
                              semd.at[sl]).wait()
        pltpu.make_async_copy(adj_hbm.at[pl.ds(off, CH)], srcb.at[sl],
                              sema.at[sl]).wait()

    def scat_wait(sl):
        pltpu.make_async_copy(rows.at[sl], acc.at[srcb.at[sl]],
                              semsc.at[sl]).wait()

    def step(k, sl, waitprev, knext):
        slnext = (sl + 1) % 3
        if waitprev:
            scat_wait(slnext)
        if knext is not None:
            idx_start(knext, slnext)
        idx_wait(k, sl)
        pltpu.async_copy(h_hbm.at[dstb.at[sl]], rows.at[sl],
                         semg.at[sl]).wait()
        pltpu.async_copy(rows.at[sl], acc.at[srcb.at[sl]], semsc.at[sl],
                         add=True)

    assert BASE_CHUNKS == 39
    idx_start(0, 0)
    step(0, 0, False, 1)
    step(1, 1, False, 2)

    def grp(g, carry):
        k0 = 2 + 3 * g
        step(k0, 2, True, k0 + 1)
        step(k0 + 1, 0, True, k0 + 2)
        step(k0 + 2, 1, True, k0 + 3)
        return carry

    lax.fori_loop(0, 12, grp, 0)
    step(38, 2, True, None)
    scat_wait(1)
    scat_wait(2)

    @pl.when(wid < EXTRA_TILES)
    def _():
        idx_start(BASE_CHUNKS, 0)
        idx_wait(BASE_CHUNKS, 0)
        pltpu.async_copy(h_hbm.at[dstb.at[0]], rows.at[0], semg.at[0]).wait()
        pltpu.async_copy(rows.at[0], acc.at[srcb.at[0]], semsc.at[0],
                         add=True)
        scat_wait(0)

    plsc.subcore_barrier()
    pltpu.sync_copy(acc.at[pl.ds(sub * RPT, RPT)],
                    ns_hbm.at[core, pl.ds(sub * RPT, RPT)])


def _input_proj_body(x_ref, w_ref, b_ref, o_ref):
    o_ref[...] = (jnp.dot(x_ref[...], w_ref[...],
                          preferred_element_type=jnp.float32) + b_ref[...])


def _layer_body(h_ref, ns_ref, w_ref, b_ref, g_ref, be_ref, o_ref):
    h = h_ref[...]
    ns = ns_ref[0] + ns_ref[1]
    w = w_ref[...]
    z = (jnp.dot(h, w[:H], preferred_element_type=jnp.float32)
         + jnp.dot(ns, w[H:], preferred_element_type=jnp.float32)
         + b_ref[...])
    mu = jnp.mean(z, axis=-1, keepdims=True)
    var = jnp.mean((z - mu) ** 2, axis=-1, keepdims=True)
    zn = (z - mu) * lax.rsqrt(var + 1e-5) * g_ref[...] + be_ref[...]
    o_ref[...] = jnp.maximum(zn, 0.0) + h


def _final_body(h_ref, ns_ref, w_ref, b_ref, g_ref, be_ref,
                wout_ref, bout_ref, wop_ref, bop_ref, gop_ref, beop_ref,
                o_ref):
    h = h_ref[...]
    ns = ns_ref[0] + ns_ref[1]
    w = w_ref[...]
    z = (jnp.dot(h, w[:H], preferred_element_type=jnp.float32)
         + jnp.dot(ns, w[H:], preferred_element_type=jnp.float32)
         + b_ref[...])
    mu = jnp.mean(z, axis=-1, keepdims=True)
    var = jnp.mean((z - mu) ** 2, axis=-1, keepdims=True)
    zn = (z - mu) * lax.rsqrt(var + 1e-5) * g_ref[...] + be_ref[...]
    hn = jnp.maximum(zn, 0.0) + h
    go = jnp.dot(hn, wout_ref[...], preferred_element_type=jnp.float32) + bout_ref[...]
    c = jnp.dot(go, wop_ref[...], preferred_element_type=jnp.float32) + bop_ref[...]
    mu2 = jnp.mean(c, axis=-1, keepdims=True)
    var2 = jnp.mean((c - mu2) ** 2, axis=-1, keepdims=True)
    cn = (c - mu2) * lax.rsqrt(var2 + 1e-5) * gop_ref[...] + beop_ref[...]
    o_ref[...] = jnp.maximum(cn, 0.0)


_ROWS = 1000
_GRID = N // _ROWS

_row_spec = pl.BlockSpec((_ROWS, H), lambda i: (i, 0))
_ns_spec = pl.BlockSpec((2, _ROWS, H), lambda i: (0, i, 0))
_vec_spec = pl.BlockSpec((1, H), lambda i: (0, 0))
_w_spec = pl.BlockSpec((2 * H, H), lambda i: (0, 0))
_sq_spec = pl.BlockSpec((H, H), lambda i: (0, 0))

_input_proj = pl.pallas_call(
    _input_proj_body,
    grid=(_GRID,),
    in_specs=[pl.BlockSpec((_ROWS, 8), lambda i: (i, 0)),
              pl.BlockSpec((8, H), lambda i: (0, 0)),
              _vec_spec],
    out_specs=_row_spec,
    out_shape=jax.ShapeDtypeStruct((N, H), jnp.float32),
)

_layer = pl.pallas_call(
    _layer_body,
    grid=(_GRID,),
    in_specs=[_row_spec, _ns_spec, _w_spec, _vec_spec, _vec_spec, _vec_spec],
    out_specs=_row_spec,
    out_shape=jax.ShapeDtypeStruct((N, H), jnp.float32),
)

_final = pl.pallas_call(
    _final_body,
    grid=(_GRID,),
    in_specs=[_row_spec, _ns_spec, _w_spec, _vec_spec, _vec_spec, _vec_spec,
              _sq_spec, _vec_spec, _sq_spec, _vec_spec, _vec_spec, _vec_spec],
    out_specs=_row_spec,
    out_shape=jax.ShapeDtypeStruct((N, OUT), jnp.float32),
)


def kernel(node_features, edge_index, W_in, b_in,
           W0, b0, g0, be0, W1, b1, g1, be1, W2, b2, g2, be2,
           W_out, b_out, W_op, b_op, g_op, be_op):
    src = edge_index[0]
    dst = edge_index[1]

    xp = jnp.pad(node_features, ((0, 0), (0, 2)))
    winp = jnp.pad(W_in, ((0, 2), (0, 0)))
    h = _input_proj(xp, winp, b_in.reshape(1, H))

    table = _dedup_scatter(src, dst)
    srcadj = _dedup_gather(src, dst, table)

    zrows = jnp.zeros((RPT, H), jnp.float32)
    layers = [(W0, b0, g0, be0), (W1, b1, g1, be1)]
    for (W, b, g, be) in layers:
        ns = _neighbor_sum(h, dst, srcadj, zrows)
        h = _layer(h, ns, W, b.reshape(1, H), g.reshape(1, H), be.reshape(1, H))

    ns = _neighbor_sum(h, dst, srcadj, zrows)
    out = _final(h, ns, W2, b2.reshape(1, H), g2.reshape(1, H), be2.reshape(1, H),
                 W_out, b_out.reshape(1, OUT), W_op[:OUT], b_op.reshape(1, OUT),
                 g_op.reshape(1, OUT), be_op.reshape(1, OUT))
    return out


# speedup vs baseline: 6.1749x; 1.3210x over previous
import functools

import jax
import jax.numpy as jnp
from jax import lax
from jax.experimental import pallas as pl
from jax.experimental.pallas import tpu as pltpu
from jax.experimental.pallas import tpu_sc as plsc

N = 10000
E = 160000
H = 64
OUT = 64

NTILES = 32
CH = 128
NCHUNKS = E // CH
BASE_CHUNKS = NCHUNKS // NTILES
EXTRA_TILES = NCHUNKS % NTILES
DUMMY = N
NPAD = 10112
RPT = NPAD // 16
TBL = N * N

_mesh = plsc.VectorSubcoreMesh(core_axis_name="c", subcore_axis_name="s")


def _wid():
    return lax.axis_index("c") * 16 + lax.axis_index("s")


@functools.partial(
    pl.kernel,
    out_type=jax.ShapeDtypeStruct((TBL, 16), jnp.int32),
    mesh=_mesh,
    compiler_params=pltpu.CompilerParams(use_tc_tiling_on_sc=False, needs_layout_passes=False),
    scratch_types=[
        pltpu.VMEM((3, CH), jnp.int32),
        pltpu.VMEM((3, CH), jnp.int32),
        pltpu.VMEM((3, CH), jnp.int32),
        pltpu.VMEM((3, CH, 16), jnp.int32),
        pltpu.SemaphoreType.DMA((3,)),
        pltpu.SemaphoreType.DMA((3,)),
        pltpu.SemaphoreType.DMA((3,)),
    ],
)
def _dedup_scatter(src_hbm, dst_hbm, table_hbm,
                   srcb, dstb, keyb, valb, semd, sema, semsc):
    wid = _wid()
    lane = lax.iota(jnp.int32, 16)

    def off_of(k):
        return (k * NTILES + wid) * CH

    def idx_start(k, sl):
        off = off_of(k)
        pltpu.async_copy(src_hbm.at[pl.ds(off, CH)], srcb.at[sl], semd.at[sl])
        pltpu.async_copy(dst_hbm.at[pl.ds(off, CH)], dstb.at[sl], sema.at[sl])

    def idx_wait(k, sl):
        off = off_of(k)
        pltpu.make_async_copy(src_hbm.at[pl.ds(off, CH)], srcb.at[sl],
                              semd.at[sl]).wait()
        pltpu.make_async_copy(dst_hbm.at[pl.ds(off, CH)], dstb.at[sl],
                              sema.at[sl]).wait()

    def scat_wait(sl):
        pltpu.make_async_copy(valb.at[sl], table_hbm.at[keyb.at[sl]],
                              semsc.at[sl]).wait()

    zero16 = jnp.zeros((16,), jnp.int32)

    def step(k, sl, waitprev, knext):
        slnext = (sl + 1) % 3
        if waitprev:
            scat_wait(slnext)
        if knext is not None:
            idx_start(knext, slnext)
        idx_wait(k, sl)
        ebase = (k * NTILES + wid) * CH
        for j in range(CH // 16):
            s = srcb[sl, pl.ds(j * 16, 16)]
            d = dstb[sl, pl.ds(j * 16, 16)]
            keyb[sl, pl.ds(j * 16, 16)] = s * N + d
            plsc.store_scatter(valb.at[sl], [j * 16 + lane, zero16],
                               ebase + j * 16 + lane)
        pltpu.async_copy(valb.at[sl], table_hbm.at[keyb.at[sl]], semsc.at[sl])

    assert BASE_CHUNKS == 39
    idx_start(0, 0)
    step(0, 0, False, 1)
    step(1, 1, False, 2)

    def grp(g, carry):
        k0 = 2 + 3 * g
        step(k0, 2, True, k0 + 1)
        step(k0 + 1, 0, True, k0 + 2)
        step(k0 + 2, 1, True, k0 + 3)
        return carry

    lax.fori_loop(0, 12, grp, 0)
    step(38, 2, True, None)
    scat_wait(1)
    scat_wait(2)

    @pl.when(wid < EXTRA_TILES)
    def _():
        idx_start(BASE_CHUNKS, 0)
        step(BASE_CHUNKS, 0, False, None)
        scat_wait(0)


@functools.partial(
    pl.kernel,
    out_type=jax.ShapeDtypeStruct((2, NPAD, H), jnp.float32),
    mesh=_mesh,
    compiler_params=pltpu.CompilerParams(use_tc_tiling_on_sc=False, needs_layout_passes=False),
    scratch_types=[
        pltpu.VMEM((4, CH), jnp.int32),
        pltpu.VMEM((4, CH), jnp.int32),
        pltpu.VMEM((4, CH, H), jnp.float32),
        pltpu.VMEM_SHARED((NPAD, H), jnp.float32),
        pltpu.SemaphoreType.DMA((4,)),
        pltpu.SemaphoreType.DMA((4,)),
        pltpu.SemaphoreType.DMA((4,)),
        pltpu.SemaphoreType.DMA((4,)),
    ],
)
def _neighbor_sum(h_hbm, dst_hbm, adj_hbm, zrows_hbm, ns_hbm,
                  dstb, srcb, rows, acc, semd, sema, semg, semsc):
    core = lax.axis_index("c")
    sub = lax.axis_index("s")
    wid = core * 16 + sub

    def off_of(k):
        return (k * NTILES + wid) * CH

    def idx_start(k, sl):
        off = off_of(k)
        pltpu.async_copy(dst_hbm.at[pl.ds(off, CH)], dstb.at[sl], semd.at[sl])
        pltpu.async_copy(adj_hbm.at[pl.ds(off, CH)], srcb.at[sl], sema.at[sl])

    def idx_wait(k, sl):
        off = off_of(k)
        pltpu.make_async_copy(dst_hbm.at[pl.ds(off, CH)], dstb.at[sl],
                              semd.at[sl]).wait()
        pltpu.make_async_copy(adj_hbm.at[pl.ds(off, CH)], srcb.at[sl],
                              sema.at[sl]).wait()

    def gath_start(sl):
        pltpu.async_copy(h_hbm.at[dstb.at[sl]], rows.at[sl], semg.at[sl])

    def gath_wait(sl):
        pltpu.make_async_copy(h_hbm.at[dstb.at[sl]], rows.at[sl],
                              semg.at[sl]).wait()

    def scat_start(sl):
        pltpu.async_copy(rows.at[sl], acc.at[srcb.at[sl]], semsc.at[sl],
                         add=True)

    def scat_wait(sl):
        pltpu.make_async_copy(rows.at[sl], acc.at[srcb.at[sl]],
                              semsc.at[sl]).wait()

    LAST = BASE_CHUNKS - 1

    def step(k, sl, waitprev=True, pre2=True, pre1=True):
        if waitprev:
            scat_wait((sl + 2) % 4)
        if pre2:
            idx_start(k + 2, (sl + 2) % 4)
        if pre1:
            idx_wait(k + 1, (sl + 1) % 4)
            gath_start((sl + 1) % 4)
        gath_wait(sl)
        scat_start(sl)

    assert BASE_CHUNKS == 39 and LAST == 38
    idx_start(0, 0)
    idx_start(1, 1)
    pltpu.sync_copy(zrows_hbm, acc.at[pl.ds(sub * RPT, RPT)])
    plsc.subcore_barrier()
    idx_wait(0, 0)
    gath_start(0)
    step(0, 0, waitprev=False)
    step(1, 1, waitprev=False)

    def grp(g, carry):
        k0 = 2 + 4 * g
        step(k0, 2)
        step(k0 + 1, 3)
        step(k0 + 2, 0)
        step(k0 + 3, 1)
        return carry

    lax.fori_loop(0, 8, grp, 0)
    step(34, 2)
    step(35, 3)
    step(36, 0)
    step(37, 1, pre2=False)
    step(38, 2, pre2=False, pre1=False)
    scat_wait(1)
    scat_wait(2)

    @pl.when(wid < EXTRA_TILES)
    def _():
        idx_start(BASE_CHUNKS, 0)
        idx_wait(BASE_CHUNKS, 0)
        gath_start(0)
        gath_wait(0)
        scat_start(0)
        scat_wait(0)

    plsc.subcore_barrier()
    pltpu.sync_copy(acc.at[pl.ds(sub * RPT, RPT)],
                    ns_hbm.at[core, pl.ds(sub * RPT, RPT)])


@functools.partial(
    pl.kernel,
    out_type=(jax.ShapeDtypeStruct((2, NPAD, H), jnp.float32),
              jax.ShapeDtypeStruct((E,), jnp.int32)),
    mesh=_mesh,
    compiler_params=pltpu.CompilerParams(use_tc_tiling_on_sc=False, needs_layout_passes=False),
    scratch_types=[
        pltpu.VMEM((4, CH), jnp.int32),
        pltpu.VMEM((4, CH), jnp.int32),
        pltpu.VMEM((4, CH), jnp.int32),
        pltpu.VMEM((4, CH, 16), jnp.int32),
        pltpu.VMEM((4, CH, H), jnp.float32),
        pltpu.VMEM_SHARED((NPAD, H), jnp.float32),
        pltpu.SemaphoreType.DMA((4,)),
        pltpu.SemaphoreType.DMA((4,)),
        pltpu.SemaphoreType.DMA((4,)),
        pltpu.SemaphoreType.DMA((4,)),
        pltpu.SemaphoreType.DMA((4,)),
        pltpu.SemaphoreType.DMA((4,)),
    ],
)
def _neighbor_sum_first(h_hbm, src_hbm, dst_hbm, table_hbm, zrows_hbm,
                        ns_hbm, adj_hbm,
                        srcb, dstb, keyb, winb, rows, acc,
                        semd, sema, semw, semg, semsc, semst):
    core = lax.axis_index("c")
    sub = lax.axis_index("s")
    wid = core * 16 + sub
    lane = lax.iota(jnp.int32, 16)
    zero16 = jnp.zeros((16,), jnp.int32)

    def off_of(k):
        return (k * NTILES + wid) * CH

    def idx_start(k, sl):
        off = off_of(k)
        pltpu.async_copy(src_hbm.at[pl.ds(off, CH)], srcb.at[sl], semd.at[sl])
        pltpu.async_copy(dst_hbm.at[pl.ds(off, CH)], dstb.at[sl], sema.at[sl])

    def idx_wait(k, sl):
        off = off_of(k)
        pltpu.make_async_copy(src_hbm.at[pl.ds(off, CH)], srcb.at[sl],
                              semd.at[sl]).wait()
        pltpu.make_async_copy(dst_hbm.at[pl.ds(off, CH)], dstb.at[sl],
                              sema.at[sl]).wait()

    def fronts(k, sl):
        for j in range(CH // 16):
            s = srcb[sl, pl.ds(j * 16, 16)]
            d = dstb[sl, pl.ds(j * 16, 16)]
            keyb[sl, pl.ds(j * 16, 16)] = s * N + d
        pltpu.async_copy(table_hbm.at[keyb.at[sl]], winb.at[sl], semw.at[sl])
        pltpu.async_copy(h_hbm.at[dstb.at[sl]], rows.at[sl], semg.at[sl])

    def resolve(k, sl):
        pltpu.make_async_copy(table_hbm.at[keyb.at[sl]], winb.at[sl],
                              semw.at[sl]).wait()
        ebase = (k * NTILES + wid) * CH
        for j in range(CH // 16):
            s = srcb[sl, pl.ds(j * 16, 16)]
            w = plsc.load_gather(winb.at[sl], [j * 16 + lane, zero16])
            e = ebase + j * 16 + lane
            srcb[sl, pl.ds(j * 16, 16)] = jnp.where(w == e, s, DUMMY)

    def finish(k, sl):
        resolve(k, sl)
        pltpu.make_async_copy(h_hbm.at[dstb.at[sl]], rows.at[sl],
                              semg.at[sl]).wait()
        pltpu.async_copy(rows.at[sl], acc.at[srcb.at[sl]], semsc.at[sl],
                         add=True)
        pltpu.async_copy(srcb.at[sl], adj_hbm.at[pl.ds(off_of(k), CH)],
                         semst.at[sl])

    def tail_wait(k, sl):
        pltpu.make_async_copy(rows.at[sl], acc.at[srcb.at[sl]],
                              semsc.at[sl]).wait()
        pltpu.make_async_copy(srcb.at[sl], adj_hbm.at[pl.ds(off_of(k), CH)],
                              semst.at[sl]).wait()

    def step(k, sl, waitprev=True, pre2=True, pre1=True):
        if waitprev:
            tail_wait(k - 2, (sl + 2) % 4)
        if pre2:
            idx_start(k + 2, (sl + 2) % 4)
        if pre1:
            idx_wait(k + 1, (sl + 1) % 4)
            fronts(k + 1, (sl + 1) % 4)
        finish(k, sl)

    assert BASE_CHUNKS == 39
    idx_start(0, 0)
    idx_start(1, 1)
    pltpu.sync_copy(zrows_hbm, acc.at[pl.ds(sub * RPT, RPT)])
    plsc.subcore_barrier()
    idx_wait(0, 0)
    fronts(0, 0)
    step(0, 0, waitprev=False)
    step(1, 1, waitprev=False)

    def grp(g, carry):
        k0 = 2 + 4 * g
        step(k0, 2)
        step(k0 + 1, 3)
        step(k0 + 2, 0)
        step(k0 + 3, 1)
        return carry

    lax.fori_loop(0, 8, grp, 0)
    step(34, 2)
    step(35, 3)
    step(36, 0)
    step(37, 1, pre2=False)
    step(38, 2, pre2=False, pre1=False)
    tail_wait(37, 1)
    tail_wait(38, 2)

    @pl.when(wid < EXTRA_TILES)
    def _():
        idx_start(BASE_CHUNKS, 0)
        idx_wait(BASE_CHUNKS, 0)
        fronts(BASE_CHUNKS, 0)
        finish(BASE_CHUNKS, 0)
        tail_wait(BASE_CHUNKS, 0)

    plsc.subcore_barrier()
    pltpu.sync_copy(acc.at[pl.ds(sub * RPT, RPT)],
                    ns_hbm.at[core, pl.ds(sub * RPT, RPT)])


def _input_proj_body(x_ref, w_ref, b_ref, o_ref):
    o_ref[...] = (jnp.dot(x_ref[...], w_ref[...],
                          preferred_element_type=jnp.float32) + b_ref[...])


def _layer_body(h_ref, ns_ref, w_ref, b_ref, g_ref, be_ref, o_ref):
    h = h_ref[...]
    ns = ns_ref[0] + ns_ref[1]
    w = w_ref[...]
    z = (jnp.dot(h, w[:H], preferred_element_type=jnp.float32)
         + jnp.dot(ns, w[H:], preferred_element_type=jnp.float32)
         + b_ref[...])
    mu = jnp.mean(z, axis=-1, keepdims=True)
    var = jnp.mean((z - mu) ** 2, axis=-1, keepdims=True)
    zn = (z - mu) * lax.rsqrt(var + 1e-5) * g_ref[...] + be_ref[...]
    o_ref[...] = jnp.maximum(zn, 0.0) + h


def _final_body(h_ref, ns_ref, w_ref, b_ref, g_ref, be_ref,
                wout_ref, bout_ref, wop_ref, bop_ref, gop_ref, beop_ref,
                o_ref):
    h = h_ref[...]
    ns = ns_ref[0] + ns_ref[1]
    w = w_ref[...]
    z = (jnp.dot(h, w[:H], preferred_element_type=jnp.float32)
         + jnp.dot(ns, w[H:], preferred_element_type=jnp.float32)
         + b_ref[...])
    mu = jnp.mean(z, axis=-1, keepdims=True)
    var = jnp.mean((z - mu) ** 2, axis=-1, keepdims=True)
    zn = (z - mu) * lax.rsqrt(var + 1e-5) * g_ref[...] + be_ref[...]
    hn = jnp.maximum(zn, 0.0) + h
    go = jnp.dot(hn, wout_ref[...], preferred_element_type=jnp.float32) + bout_ref[...]
    c = jnp.dot(go, wop_ref[...], preferred_element_type=jnp.float32) + bop_ref[...]
    mu2 = jnp.mean(c, axis=-1, keepdims=True)
    var2 = jnp.mean((c - mu2) ** 2, axis=-1, keepdims=True)
    cn = (c - mu2) * lax.rsqrt(var2 + 1e-5) * gop_ref[...] + beop_ref[...]
    o_ref[...] = jnp.maximum(cn, 0.0)


_ROWS = 2000
_GRID = N // _ROWS

_row_spec = pl.BlockSpec((_ROWS, H), lambda i: (i, 0))
_ns_spec = pl.BlockSpec((2, _ROWS, H), lambda i: (0, i, 0))
_vec_spec = pl.BlockSpec((1, H), lambda i: (0, 0))
_w_spec = pl.BlockSpec((2 * H, H), lambda i: (0, 0))
_sq_spec = pl.BlockSpec((H, H), lambda i: (0, 0))

_input_proj = pl.pallas_call(
    _input_proj_body,
    grid=(_GRID,),
    in_specs=[pl.BlockSpec((_ROWS, 8), lambda i: (i, 0)),
              pl.BlockSpec((8, H), lambda i: (0, 0)),
              _vec_spec],
    out_specs=_row_spec,
    out_shape=jax.ShapeDtypeStruct((N, H), jnp.float32),
)

_layer = pl.pallas_call(
    _layer_body,
    grid=(_GRID,),
    in_specs=[_row_spec, _ns_spec, _w_spec, _vec_spec, _vec_spec, _vec_spec],
    out_specs=_row_spec,
    out_shape=jax.ShapeDtypeStruct((N, H), jnp.float32),
)

_final = pl.pallas_call(
    _final_body,
    grid=(_GRID,),
    in_specs=[_row_spec, _ns_spec, _w_spec, _vec_spec, _vec_spec, _vec_spec,
              _sq_spec, _vec_spec, _sq_spec, _vec_spec, _vec_spec, _vec_spec],
    out_specs=_row_spec,
    out_shape=jax.ShapeDtypeStruct((N, OUT), jnp.float32),
)


def kernel(node_features, edge_index, W_in, b_in,
           W0, b0, g0, be0, W1, b1, g1, be1, W2, b2, g2, be2,
           W_out, b_out, W_op, b_op, g_op, be_op):
    src = edge_index[0]
    dst = edge_index[1]

    xp = jnp.pad(node_features, ((0, 0), (0, 2)))
    winp = jnp.pad(W_in, ((0, 2), (0, 0)))
    h = _input_proj(xp, winp, b_in.reshape(1, H))

    table = _dedup_scatter(src, dst)

    zrows = jnp.zeros((RPT, H), jnp.float32)
    ns, srcadj = _neighbor_sum_first(h, src, dst, table, zrows)
    h = _layer(h, ns, W0, b0.reshape(1, H), g0.reshape(1, H),
               be0.reshape(1, H))

    for (W, b, g, be) in [(W1, b1, g1, be1)]:
        ns = _neighbor_sum(h, dst, srcadj, zrows)
        h = _layer(h, ns, W, b.reshape(1, H), g.reshape(1, H), be.reshape(1, H))

    ns = _neighbor_sum(h, dst, srcadj, zrows)
    out = _final(h, ns, W2, b2.reshape(1, H), g2.reshape(1, H), be2.reshape(1, H),
                 W_out, b_out.reshape(1, OUT), W_op[:OUT], b_op.reshape(1, OUT),
                 g_op.reshape(1, OUT), be_op.reshape(1, OUT))
    return out
